# baseline (speedup 1.0000x reference)
import functools

import jax
import jax.numpy as jnp
from jax import lax
from jax.experimental import pallas as pl
from jax.experimental.pallas import tpu as pltpu
from jax.experimental.pallas import tpu_sc as plsc

N = 10000
E = 160000
B = 16
NF = 128
EF = 16
GF = 16
H = 512

NC = 2
NS = 16
NW = NC * NS
C = 128
EP = 163840
CPW = EP // NW // C
NP = 10240
STRIPE = NP // NS

NA = NF + B
TE = 1280
TN = 2048

_f32 = jnp.float32
_i32 = jnp.int32

_MESH = plsc.VectorSubcoreMesh(core_axis_name="c", subcore_axis_name="s")


def _sc_gather(xp, row3, col3, zeros_nf):
    @functools.partial(
        pl.kernel, mesh=_MESH,
        out_type=[
            jax.ShapeDtypeStruct((EP, NF), _f32),
            jax.ShapeDtypeStruct((EP, NF), _f32),
            jax.ShapeDtypeStruct((NC, NP, NF), _f32),
        ],
        scratch_types=[
            pltpu.VMEM((CPW, C), _i32), pltpu.VMEM((CPW, C), _i32),
            pltpu.VMEM((C, NF), _f32), pltpu.VMEM((C, NF), _f32),
            pltpu.VMEM_SHARED((NP, NF), _f32),
            pltpu.SemaphoreType.DMA, pltpu.SemaphoreType.DMA,
            pltpu.SemaphoreType.DMA, pltpu.SemaphoreType.DMA,
        ],
    )
    def k(x_h, row_h, col_h, z_h, xr_h, xc_h, sx_h,
          ri_v, ci_v, xr_v, xc_v, acc_s, sgr, sgc, swr, swc):
        cid = lax.axis_index("c")
        sid = lax.axis_index("s")
        wid = sid * NC + cid
        pltpu.sync_copy(z_h, xr_v)
        for z in range(STRIPE // C):
            pltpu.sync_copy(xr_v, acc_s.at[pl.ds(sid * STRIPE + z * C, C)])
        pltpu.sync_copy(row_h.at[wid], ri_v)
        pltpu.sync_copy(col_h.at[wid], ci_v)
        plsc.subcore_barrier()

        def fetch(off):
            pltpu.async_copy(x_h.at[ri_v.at[off]], xr_v, sgr)
            pltpu.async_copy(x_h.at[ci_v.at[off]], xc_v, sgc)

        fetch(0)

        def body(i, carry):
            base = wid * (EP // NW) + i * C
            pltpu.make_async_copy(x_h.at[ri_v.at[i]], xr_v, sgr).wait()
            wr = pltpu.async_copy(xr_v, xr_h.at[pl.ds(base, C)], swr)
            pltpu.sync_copy(xr_v, acc_s.at[ci_v.at[i]], add=True)
            pltpu.make_async_copy(x_h.at[ci_v.at[i]], xc_v, sgc).wait()
            wc = pltpu.async_copy(xc_v, xc_h.at[pl.ds(base, C)], swc)
            wr.wait()
            wc.wait()

            @pl.when(i + 1 < CPW)
            def _():
                fetch(i + 1)
            return carry

        lax.fori_loop(0, CPW, body, 0)
        plsc.subcore_barrier()
        pltpu.sync_copy(acc_s.at[pl.ds(sid * STRIPE, STRIPE)],
                        sx_h.at[cid, pl.ds(sid * STRIPE, STRIPE)])

    return k(xp, row3, col3, zeros_nf)


def _sc_scatter_ea(eaaug, col3, zeros_nf):
    @functools.partial(
        pl.kernel, mesh=_MESH,
        out_type=jax.ShapeDtypeStruct((NC, NP, NF), _f32),
        scratch_types=[
            pltpu.VMEM((CPW, C), _i32), pltpu.VMEM((2, C, NF), _f32),
            pltpu.VMEM_SHARED((NP, NF), _f32),
            pltpu.SemaphoreType.DMA, pltpu.SemaphoreType.DMA,
        ],
    )
    def k(ea_h, col_h, z_h, sea_h, ci_v, ea_v, acc_s, se0, se1):
        cid = lax.axis_index("c")
        sid = lax.axis_index("s")
        wid = sid * NC + cid
        ses = (se0, se1)
        pltpu.sync_copy(z_h, ea_v.at[0])
        for z in range(STRIPE // C):
            pltpu.sync_copy(ea_v.at[0], acc_s.at[pl.ds(sid * STRIPE + z * C, C)])
        pltpu.sync_copy(col_h.at[wid], ci_v)
        plsc.subcore_barrier()

        def fetch(off, b):
            base = wid * (EP // NW) + off * C
            pltpu.async_copy(ea_h.at[pl.ds(base, C)], ea_v.at[b], ses[b])

        def drain(off, b):
            base = wid * (EP // NW) + off * C
            pltpu.make_async_copy(ea_h.at[pl.ds(base, C)], ea_v.at[b], ses[b]).wait()

        fetch(0, 0)
        fetch(1, 1)

        def body(j, carry):
            for b in range(2):
                off = 2 * j + b
                drain(off, b)
                pltpu.sync_copy(ea_v.at[b], acc_s.at[ci_v.at[off]], add=True)

                @pl.when(off + 2 < CPW)
                def _():
                    fetch(off + 2, b)
            return carry

        lax.fori_loop(0, CPW // 2, body, 0)
        plsc.subcore_barrier()
        pltpu.sync_copy(acc_s.at[pl.ds(sid * STRIPE, STRIPE)],
                        sea_h.at[cid, pl.ds(sid * STRIPE, STRIPE)])

    return k(eaaug, col3, zeros_nf)


def _dot(a, b):
    return lax.dot_general(a, b, (((1,), (0,)), ((), ())), preferred_element_type=_f32)


def _dotb(a, b):
    return lax.dot_general(a.astype(jnp.bfloat16), b.astype(jnp.bfloat16),
                           (((1,), (0,)), ((), ())), preferred_element_type=_f32)


def _dott(a, b):
    return lax.dot_general(a, b, (((0,), (0,)), ((), ())), preferred_element_type=_f32)


def _onehot(idx, k):
    return (idx[:, None] == lax.broadcasted_iota(_i32, (idx.shape[0], k), 1)).astype(_f32)


def _bound_onehot(r3, starts, ends):
    rt = r3[0, 0, :][:, None]
    return ((rt >= starts[...]) & (rt < ends[...])).astype(_f32)


def _edge1_body(xr, xc, ea, r3, starts, ends, Ws, Wd, We, ug, W2, b2, out):
    oh = _bound_onehot(r3, starts, ends)
    h = (_dotb(xr[...], Ws[...]) + _dotb(xc[...], Wd[...])
         + _dot(ea[...], We[...]) + _dot(oh, ug[...]))
    eao = _dot(jnp.maximum(h, 0.0), W2[...]) + b2[...]
    out[...] = jnp.concatenate(
        [eao, jnp.ones((TE, 1), _f32), jnp.zeros((TE, NF - EF - 1), _f32)], axis=1)


def _edge1(xr, xc, eattr, r3, starts, ends, Ws, Wd, We, ug, W2, b2):
    row = lambda i: (i, 0)
    full = lambda i: (0, 0)
    return pl.pallas_call(
        _edge1_body,
        grid=(EP // TE,),
        in_specs=[
            pl.BlockSpec((TE, NF), row), pl.BlockSpec((TE, NF), row),
            pl.BlockSpec((TE, EF), row),
            pl.BlockSpec((1, 1, TE), lambda i: (i, 0, 0)),
            pl.BlockSpec((1, B), full), pl.BlockSpec((1, B), full),
            pl.BlockSpec((NF, H), full), pl.BlockSpec((NF, H), full),
            pl.BlockSpec((EF, H), full), pl.BlockSpec((B, H), full),
            pl.BlockSpec((H, EF), full), pl.BlockSpec((1, EF), full),
        ],
        out_specs=pl.BlockSpec((TE, NF), row),
        out_shape=jax.ShapeDtypeStruct((EP, NF), _f32),
    )(xr, xc, eattr, r3, starts, ends, Ws, Wd, We, ug, W2, b2)


def _edge2_body(xr, xr1, xc, xc1, ea, eaaug1, r3, starts, ends,
                Wsx, Wsy, Wdx, Wdy, Wee, Wea, ug, W2, b2, out):
    oh = _bound_onehot(r3, starts, ends)
    h = (_dotb(xr[...], Wsx[...]) + _dotb(xr1[...], Wsy[...])
         + _dotb(xc[...], Wdx[...]) + _dotb(xc1[...], Wdy[...])
         + _dot(ea[...], Wee[...]) + _dot(eaaug1[:, :EF], Wea[...]) + _dot(oh, ug[...]))
    eao = _dot(jnp.maximum(h, 0.0), W2[...]) + b2[...]
    out[...] = jnp.concatenate(
        [eao, jnp.ones((TE, 1), _f32), jnp.zeros((TE, NF - EF - 1), _f32)], axis=1)


def _edge2(xr, xr1, xc, xc1, eattr, eaaug1, r3, starts, ends,
           Wsx, Wsy, Wdx, Wdy, Wee, Wea, ug, W2, b2):
    row = lambda i: (i, 0)
    full = lambda i: (0, 0)
    return pl.pallas_call(
        _edge2_body,
        grid=(EP // TE,),
        in_specs=[
            pl.BlockSpec((TE, NF), row), pl.BlockSpec((TE, NF), row),
            pl.BlockSpec((TE, NF), row), pl.BlockSpec((TE, NF), row),
            pl.BlockSpec((TE, EF), row), pl.BlockSpec((TE, NF), row),
            pl.BlockSpec((1, 1, TE), lambda i: (i, 0, 0)),
            pl.BlockSpec((1, B), full), pl.BlockSpec((1, B), full),
            pl.BlockSpec((NF, H), full), pl.BlockSpec((NF, H), full),
            pl.BlockSpec((NF, H), full), pl.BlockSpec((NF, H), full),
            pl.BlockSpec((EF, H), full), pl.BlockSpec((EF, H), full),
            pl.BlockSpec((B, H), full), pl.BlockSpec((H, EF), full),
            pl.BlockSpec((1, EF), full),
        ],
        out_specs=pl.BlockSpec((TE, NF), row),
        out_shape=jax.ShapeDtypeStruct((EP, NF), _f32),
    )(xr, xr1, xc, xc1, eattr, eaaug1, r3, starts, ends,
      Wsx, Wsy, Wdx, Wdy, Wee, Wea, ug, W2, b2)


def _node1_body(x, Sxp, Seap, b3, u, m1Wx, m1We, m1b, W1x, W1a, W1u, b1, W2, b2,
                xn_out, xsum_out):
    i = pl.program_id(0)
    Seac = Seap[0] + Seap[1]
    Sx = Sxp[0] + Sxp[1]
    Sea = Seac[:, :EF]
    cnt = Seac[:, EF:EF + 1]
    cntc = jnp.maximum(cnt, 1.0)
    agg = (_dot(Sx, m1Wx[...]) + _dot(Sea, m1We[...]) + cnt * m1b[...]) / cntc
    oh = _onehot(b3[0, 0, :], B)
    ub = _dot(oh, u[...])
    h = jnp.maximum(_dot(x[...], W1x[...]) + _dot(agg, W1a[...]) + _dot(ub, W1u[...]) + b1[...], 0.0)
    xn = _dot(h, W2[...]) + b2[...]
    xn_out[...] = xn

    @pl.when(i == 0)
    def _():
        xsum_out[...] = jnp.zeros_like(xsum_out)

    xsum_out[...] += _dott(oh, xn)


def _node1(x, Sxp, Seap, b3, u, m1Wx, m1We, m1b, W1x, W1a, W1u, b1, W2, b2):
    row = lambda i: (i, 0)
    row3 = lambda i: (0, i, 0)
    full = lambda i: (0, 0)
    return pl.pallas_call(
        _node1_body,
        grid=(NP // TN,),
        in_specs=[
            pl.BlockSpec((TN, NF), row), pl.BlockSpec((NC, TN, NF), row3),
            pl.BlockSpec((NC, TN, NF), row3), pl.BlockSpec((1, 1, TN), lambda i: (i, 0, 0)),
            pl.BlockSpec((B, GF), full),
            pl.BlockSpec((NF, H), full), pl.BlockSpec((EF, H), full),
            pl.BlockSpec((1, H), full),
            pl.BlockSpec((NF, H), full), pl.BlockSpec((H, H), full),
            pl.BlockSpec((GF, H), full), pl.BlockSpec((1, H), full),
            pl.BlockSpec((H, NF), full), pl.BlockSpec((1, NF), full),
        ],
        out_specs=[pl.BlockSpec((TN, NF), row), pl.BlockSpec((B, NF), full)],
        out_shape=[jax.ShapeDtypeStruct((NP, NF), _f32),
                   jax.ShapeDtypeStruct((B, NF), _f32)],
    )(x, Sxp, Seap, b3, u, m1Wx, m1We, m1b, W1x, W1a, W1u, b1, W2, b2)


def _node2_body(x, x1, Sxp, Sx1p, Seap, b3, uc, m1Wx, m1Wy, m1We, m1b,
                W1x, W1y, W1a, W1u, b1, W2, b2, xn_out):
    Seac = Seap[0] + Seap[1]
    Sx = Sxp[0] + Sxp[1]
    Sx1 = Sx1p[0] + Sx1p[1]
    Sea = Seac[:, :EF]
    cnt = Seac[:, EF:EF + 1]
    cntc = jnp.maximum(cnt, 1.0)
    agg = (_dot(Sx, m1Wx[...]) + _dot(Sx1, m1Wy[...])
           + _dot(Sea, m1We[...]) + cnt * m1b[...]) / cntc
    oh = _onehot(b3[0, 0, :], B)
    ub = _dot(oh, uc[...])
    h = jnp.maximum(_dot(x[...], W1x[...]) + _dot(x1[...], W1y[...])
                    + _dot(agg, W1a[...]) + _dot(ub, W1u[...]) + b1[...], 0.0)
    xn_out[...] = _dot(h, W2[...]) + b2[...]


def _node2(x, x1, Sxp, Sx1p, Seap, b3, uc, m1Wx, m1Wy, m1We, m1b,
           W1x, W1y, W1a, W1u, b1, W2, b2):
    row = lambda i: (i, 0)
    row3 = lambda i: (0, i, 0)
    full = lambda i: (0, 0)
    return pl.pallas_call(
        _node2_body,
        grid=(NP // TN,),
        in_specs=[
            pl.BlockSpec((TN, NF), row), pl.BlockSpec((TN, NF), row),
            pl.BlockSpec((NC, TN, NF), row3), pl.BlockSpec((NC, TN, NF), row3),
            pl.BlockSpec((NC, TN, NF), row3), pl.BlockSpec((1, 1, TN), lambda i: (i, 0, 0)),
            pl.BlockSpec((B, 2 * GF), full),
            pl.BlockSpec((NF, H), full), pl.BlockSpec((NF, H), full),
            pl.BlockSpec((EF, H), full), pl.BlockSpec((1, H), full),
            pl.BlockSpec((NF, H), full), pl.BlockSpec((NF, H), full),
            pl.BlockSpec((H, H), full), pl.BlockSpec((2 * GF, H), full),
            pl.BlockSpec((1, H), full),
            pl.BlockSpec((H, NF), full), pl.BlockSpec((1, NF), full),
        ],
        out_specs=pl.BlockSpec((TN, NF), row),
        out_shape=jax.ShapeDtypeStruct((NP, NF), _f32),
    )(x, x1, Sxp, Sx1p, Seap, b3, uc, m1Wx, m1Wy, m1We, m1b,
      W1x, W1y, W1a, W1u, b1, W2, b2)


def _padn(a):
    return jnp.pad(a, ((0, NP - N),) + ((0, 0),) * (a.ndim - 1))


def kernel(x, edge_index, edge_attr, u, batch, e1_W1, e1_b1, e1_W2, e1_b2,
           n1_m1_W, n1_m1_b, n1_m2_W1, n1_m2_b1, n1_m2_W2, n1_m2_b2,
           g1_W1, g1_b1, g1_W2, g1_b2, e2_W1, e2_b1, e2_W2, e2_b2,
           n2_m1_W, n2_m1_b, n2_m2_W1, n2_m2_b1, n2_m2_W2, n2_m2_b2,
           g2_W1, g2_b1, g2_W2, g2_b2):
    row, col = edge_index[0], edge_index[1]
    rowp = jnp.pad(row, (0, EP - E))
    colp = jnp.pad(col, (0, EP - E), constant_values=NP - C)
    eattrp = jnp.pad(edge_attr, ((0, EP - E), (0, 0)))
    batchp = jnp.pad(batch, (0, NP - N), constant_values=B)
    zeros_nf = jnp.zeros((C, NF), _f32)
    xp = _padn(x)
    starts = jnp.searchsorted(batch, jnp.arange(B, dtype=_i32)).astype(_i32).reshape(1, B)
    ends = jnp.searchsorted(batch, jnp.arange(B, dtype=_i32), side="right").astype(_i32).reshape(1, B)
    r3 = rowp.reshape(EP // TE, 1, TE)
    row3 = rowp.reshape(NW, CPW, C)
    col3 = colp.reshape(NW, CPW, C)

    xr, xc, Sxp = _sc_gather(xp, row3, col3, zeros_nf)

    ug1 = u @ e1_W1[2 * NF + EF:] + e1_b1
    eaaug1 = _edge1(xr, xc, eattrp, r3, starts, ends,
                    e1_W1[:NF], e1_W1[NF:2 * NF], e1_W1[2 * NF:2 * NF + EF], ug1,
                    e1_W2, e1_b2.reshape(1, EF))

    Seap1 = _sc_scatter_ea(eaaug1, col3, zeros_nf)

    b3 = batchp.reshape(NP // TN, 1, TN)
    x1p, xsum = _node1(xp, Sxp, Seap1, b3, u,
                       n1_m1_W[:NF], n1_m1_W[NF:], n1_m1_b.reshape(1, H),
                       n1_m2_W1[:NF], n1_m2_W1[NF:NF + H], n1_m2_W1[NF + H:],
                       n1_m2_b1.reshape(1, H), n1_m2_W2, n1_m2_b2.reshape(1, NF))

    bcnt = jnp.maximum((ends - starts).astype(_f32).reshape(B), 1.0)
    xmean = xsum / bcnt[:, None]
    gh = jnp.maximum(jnp.concatenate([u, xmean], axis=1) @ g1_W1 + g1_b1, 0.0)
    u1 = gh @ g1_W2 + g1_b2
    uc = jnp.concatenate([u, u1], axis=1)

    xr1, xc1, Sx1p = _sc_gather(x1p, row3, col3, zeros_nf)

    ug2 = uc @ e2_W1[4 * NF + 2 * EF:] + e2_b1
    eaaug2 = _edge2(xr, xr1, xc, xc1, eattrp, eaaug1, r3, starts, ends,
                    e2_W1[:NF], e2_W1[NF:2 * NF], e2_W1[2 * NF:3 * NF], e2_W1[3 * NF:4 * NF],
                    e2_W1[4 * NF:4 * NF + EF], e2_W1[4 * NF + EF:4 * NF + 2 * EF], ug2,
                    e2_W2, e2_b2.reshape(1, EF))

    Seap2 = _sc_scatter_ea(eaaug2, col3, zeros_nf)

    x2p = _node2(xp, x1p, Sxp, Sx1p, Seap2, b3, uc,
                 n2_m1_W[:NF], n2_m1_W[NF:2 * NF], n2_m1_W[2 * NF:],
                 n2_m1_b.reshape(1, H),
                 n2_m2_W1[:NF], n2_m2_W1[NF:2 * NF], n2_m2_W1[2 * NF:2 * NF + H],
                 n2_m2_W1[2 * NF + H:], n2_m2_b1.reshape(1, H),
                 n2_m2_W2, n2_m2_b2.reshape(1, NF))
    return x2p[:N]


# speedup vs baseline: 1.0607x; 1.0607x over previous
import functools

import jax
import jax.numpy as jnp
from jax import lax
from jax.experimental import pallas as pl
from jax.experimental.pallas import tpu as pltpu
from jax.experimental.pallas import tpu_sc as plsc

N = 10000
E = 160000
B = 16
NF = 128
EF = 16
GF = 16
H = 512

NC = 2
NS = 16
NW = NC * NS
C = 128
EP = 163840
NH = 2
EP2 = EP // NH
CPW = EP2 // NW // C
NP = 10240
STRIPE = NP // NS

NA = NF + B
TE = 1280
TN = 2048

_f32 = jnp.float32
_i32 = jnp.int32

_MESH = plsc.VectorSubcoreMesh(core_axis_name="c", subcore_axis_name="s")


def _sc_gather(xp, row3, col3, zeros_nf):
    @functools.partial(
        pl.kernel, mesh=_MESH,
        out_type=[
            jax.ShapeDtypeStruct((EP2, NF), _f32),
            jax.ShapeDtypeStruct((EP2, NF), _f32),
            jax.ShapeDtypeStruct((NC, NP, NF), _f32),
        ],
        scratch_types=[
            pltpu.VMEM((CPW, C), _i32), pltpu.VMEM((CPW, C), _i32),
            pltpu.VMEM((C, NF), _f32), pltpu.VMEM((C, NF), _f32),
            pltpu.VMEM_SHARED((NP, NF), _f32),
            pltpu.SemaphoreType.DMA, pltpu.SemaphoreType.DMA,
            pltpu.SemaphoreType.DMA, pltpu.SemaphoreType.DMA,
        ],
    )
    def k(x_h, row_h, col_h, z_h, xr_h, xc_h, sx_h,
          ri_v, ci_v, xr_v, xc_v, acc_s, sgr, sgc, swr, swc):
        cid = lax.axis_index("c")
        sid = lax.axis_index("s")
        wid = sid * NC + cid
        pltpu.sync_copy(z_h, xr_v)
        for z in range(STRIPE // C):
            pltpu.sync_copy(xr_v, acc_s.at[pl.ds(sid * STRIPE + z * C, C)])
        pltpu.sync_copy(row_h.at[wid], ri_v)
        pltpu.sync_copy(col_h.at[wid], ci_v)
        plsc.subcore_barrier()

        def fetch(off):
            pltpu.async_copy(x_h.at[ri_v.at[off]], xr_v, sgr)
            pltpu.async_copy(x_h.at[ci_v.at[off]], xc_v, sgc)

        fetch(0)

        def body(i, carry):
            base = wid * (EP2 // NW) + i * C
            pltpu.make_async_copy(x_h.at[ri_v.at[i]], xr_v, sgr).wait()
            wr = pltpu.async_copy(xr_v, xr_h.at[pl.ds(base, C)], swr)
            pltpu.sync_copy(xr_v, acc_s.at[ci_v.at[i]], add=True)
            pltpu.make_async_copy(x_h.at[ci_v.at[i]], xc_v, sgc).wait()
            wc = pltpu.async_copy(xc_v, xc_h.at[pl.ds(base, C)], swc)
            wr.wait()
            wc.wait()

            @pl.when(i + 1 < CPW)
            def _():
                fetch(i + 1)
            return carry

        lax.fori_loop(0, CPW, body, 0)
        plsc.subcore_barrier()
        pltpu.sync_copy(acc_s.at[pl.ds(sid * STRIPE, STRIPE)],
                        sx_h.at[cid, pl.ds(sid * STRIPE, STRIPE)])

    return k(xp, row3, col3, zeros_nf)


def _sc_scatter_ea(eaaug, col3, zeros_nf):
    @functools.partial(
        pl.kernel, mesh=_MESH,
        out_type=jax.ShapeDtypeStruct((NC, NP, NF), _f32),
        scratch_types=[
            pltpu.VMEM((CPW, C), _i32), pltpu.VMEM((2, C, NF), _f32),
            pltpu.VMEM_SHARED((NP, NF), _f32),
            pltpu.SemaphoreType.DMA, pltpu.SemaphoreType.DMA,
        ],
    )
    def k(ea_h, col_h, z_h, sea_h, ci_v, ea_v, acc_s, se0, se1):
        cid = lax.axis_index("c")
        sid = lax.axis_index("s")
        wid = sid * NC + cid
        ses = (se0, se1)
        pltpu.sync_copy(z_h, ea_v.at[0])
        for z in range(STRIPE // C):
            pltpu.sync_copy(ea_v.at[0], acc_s.at[pl.ds(sid * STRIPE + z * C, C)])
        pltpu.sync_copy(col_h.at[wid], ci_v)
        plsc.subcore_barrier()

        def fetch(off, b):
            base = wid * (EP2 // NW) + off * C
            pltpu.async_copy(ea_h.at[pl.ds(base, C)], ea_v.at[b], ses[b])

        def drain(off, b):
            base = wid * (EP2 // NW) + off * C
            pltpu.make_async_copy(ea_h.at[pl.ds(base, C)], ea_v.at[b], ses[b]).wait()

        fetch(0, 0)
        fetch(1, 1)

        def body(j, carry):
            for b in range(2):
                off = 2 * j + b
                drain(off, b)
                pltpu.sync_copy(ea_v.at[b], acc_s.at[ci_v.at[off]], add=True)

                @pl.when(off + 2 < CPW)
                def _():
                    fetch(off + 2, b)
            return carry

        lax.fori_loop(0, CPW // 2, body, 0)
        plsc.subcore_barrier()
        pltpu.sync_copy(acc_s.at[pl.ds(sid * STRIPE, STRIPE)],
                        sea_h.at[cid, pl.ds(sid * STRIPE, STRIPE)])

    return k(eaaug, col3, zeros_nf)


def _dot(a, b):
    return lax.dot_general(a, b, (((1,), (0,)), ((), ())), preferred_element_type=_f32)


def _dotb(a, b):
    return lax.dot_general(a.astype(jnp.bfloat16), b.astype(jnp.bfloat16),
                           (((1,), (0,)), ((), ())), preferred_element_type=_f32)


def _dott(a, b):
    return lax.dot_general(a, b, (((0,), (0,)), ((), ())), preferred_element_type=_f32)


def _onehot(idx, k):
    return (idx[:, None] == lax.broadcasted_iota(_i32, (idx.shape[0], k), 1)).astype(_f32)


def _bound_onehot(r3, starts, ends):
    rt = r3[0, 0, :][:, None]
    return ((rt >= starts[...]) & (rt < ends[...])).astype(_f32)


def _edge1_body(xr, xc, ea, r3, starts, ends, Ws, Wd, We, ug, W2, b2, out):
    oh = _bound_onehot(r3, starts, ends)
    h = (_dotb(xr[...], Ws[...]) + _dotb(xc[...], Wd[...])
         + _dot(ea[...], We[...]) + _dot(oh, ug[...]))
    eao = _dot(jnp.maximum(h, 0.0), W2[...]) + b2[...]
    out[...] = jnp.concatenate(
        [eao, jnp.ones((TE, 1), _f32), jnp.zeros((TE, NF - EF - 1), _f32)], axis=1)


def _edge1(xr, xc, eattr, r3, starts, ends, Ws, Wd, We, ug, W2, b2):
    row = lambda i: (i, 0)
    full = lambda i: (0, 0)
    return pl.pallas_call(
        _edge1_body,
        grid=(EP2 // TE,),
        in_specs=[
            pl.BlockSpec((TE, NF), row), pl.BlockSpec((TE, NF), row),
            pl.BlockSpec((TE, EF), row),
            pl.BlockSpec((1, 1, TE), lambda i: (i, 0, 0)),
            pl.BlockSpec((1, B), full), pl.BlockSpec((1, B), full),
            pl.BlockSpec((NF, H), full), pl.BlockSpec((NF, H), full),
            pl.BlockSpec((EF, H), full), pl.BlockSpec((B, H), full),
            pl.BlockSpec((H, EF), full), pl.BlockSpec((1, EF), full),
        ],
        out_specs=pl.BlockSpec((TE, NF), row),
        out_shape=jax.ShapeDtypeStruct((EP2, NF), _f32),
    )(xr, xc, eattr, r3, starts, ends, Ws, Wd, We, ug, W2, b2)


def _edge2_body(xr, xr1, xc, xc1, ea, eaaug1, r3, starts, ends,
                Wsx, Wsy, Wdx, Wdy, Wee, Wea, ug, W2, b2, out):
    oh = _bound_onehot(r3, starts, ends)
    h = (_dotb(xr[...], Wsx[...]) + _dotb(xr1[...], Wsy[...])
         + _dotb(xc[...], Wdx[...]) + _dotb(xc1[...], Wdy[...])
         + _dot(ea[...], Wee[...]) + _dot(eaaug1[:, :EF], Wea[...]) + _dot(oh, ug[...]))
    eao = _dot(jnp.maximum(h, 0.0), W2[...]) + b2[...]
    out[...] = jnp.concatenate(
        [eao, jnp.ones((TE, 1), _f32), jnp.zeros((TE, NF - EF - 1), _f32)], axis=1)


def _edge2(xr, xr1, xc, xc1, eattr, eaaug1, r3, starts, ends,
           Wsx, Wsy, Wdx, Wdy, Wee, Wea, ug, W2, b2):
    row = lambda i: (i, 0)
    full = lambda i: (0, 0)
    return pl.pallas_call(
        _edge2_body,
        grid=(EP2 // TE,),
        in_specs=[
            pl.BlockSpec((TE, NF), row), pl.BlockSpec((TE, NF), row),
            pl.BlockSpec((TE, NF), row), pl.BlockSpec((TE, NF), row),
            pl.BlockSpec((TE, EF), row), pl.BlockSpec((TE, NF), row),
            pl.BlockSpec((1, 1, TE), lambda i: (i, 0, 0)),
            pl.BlockSpec((1, B), full), pl.BlockSpec((1, B), full),
            pl.BlockSpec((NF, H), full), pl.BlockSpec((NF, H), full),
            pl.BlockSpec((NF, H), full), pl.BlockSpec((NF, H), full),
            pl.BlockSpec((EF, H), full), pl.BlockSpec((EF, H), full),
            pl.BlockSpec((B, H), full), pl.BlockSpec((H, EF), full),
            pl.BlockSpec((1, EF), full),
        ],
        out_specs=pl.BlockSpec((TE, NF), row),
        out_shape=jax.ShapeDtypeStruct((EP2, NF), _f32),
    )(xr, xr1, xc, xc1, eattr, eaaug1, r3, starts, ends,
      Wsx, Wsy, Wdx, Wdy, Wee, Wea, ug, W2, b2)


def _node1_body(x, Sxp, Sxq, Seap, Seaq, b3, u, m1Wx, m1We, m1b, W1x, W1a, W1u, b1, W2, b2,
                xn_out, xsum_out):
    i = pl.program_id(0)
    Seac = Seap[0] + Seap[1] + Seaq[0] + Seaq[1]
    Sx = Sxp[0] + Sxp[1] + Sxq[0] + Sxq[1]
    Sea = Seac[:, :EF]
    cnt = Seac[:, EF:EF + 1]
    cntc = jnp.maximum(cnt, 1.0)
    agg = (_dot(Sx, m1Wx[...]) + _dot(Sea, m1We[...]) + cnt * m1b[...]) / cntc
    oh = _onehot(b3[0, 0, :], B)
    ub = _dot(oh, u[...])
    h = jnp.maximum(_dot(x[...], W1x[...]) + _dot(agg, W1a[...]) + _dot(ub, W1u[...]) + b1[...], 0.0)
    xn = _dot(h, W2[...]) + b2[...]
    xn_out[...] = xn

    @pl.when(i == 0)
    def _():
        xsum_out[...] = jnp.zeros_like(xsum_out)

    xsum_out[...] += _dott(oh, xn)


def _node1(x, Sxp, Sxq, Seap, Seaq, b3, u, m1Wx, m1We, m1b, W1x, W1a, W1u, b1, W2, b2):
    row = lambda i: (i, 0)
    row3 = lambda i: (0, i, 0)
    full = lambda i: (0, 0)
    return pl.pallas_call(
        _node1_body,
        grid=(NP // TN,),
        in_specs=[
            pl.BlockSpec((TN, NF), row), pl.BlockSpec((NC, TN, NF), row3),
            pl.BlockSpec((NC, TN, NF), row3), pl.BlockSpec((NC, TN, NF), row3),
            pl.BlockSpec((NC, TN, NF), row3), pl.BlockSpec((1, 1, TN), lambda i: (i, 0, 0)),
            pl.BlockSpec((B, GF), full),
            pl.BlockSpec((NF, H), full), pl.BlockSpec((EF, H), full),
            pl.BlockSpec((1, H), full),
            pl.BlockSpec((NF, H), full), pl.BlockSpec((H, H), full),
            pl.BlockSpec((GF, H), full), pl.BlockSpec((1, H), full),
            pl.BlockSpec((H, NF), full), pl.BlockSpec((1, NF), full),
        ],
        out_specs=[pl.BlockSpec((TN, NF), row), pl.BlockSpec((B, NF), full)],
        out_shape=[jax.ShapeDtypeStruct((NP, NF), _f32),
                   jax.ShapeDtypeStruct((B, NF), _f32)],
    )(x, Sxp, Sxq, Seap, Seaq, b3, u, m1Wx, m1We, m1b, W1x, W1a, W1u, b1, W2, b2)


def _node2_body(x, x1, Sxp, Sxq, Sx1p, Sx1q, Seap, Seaq, b3, uc, m1Wx, m1Wy, m1We, m1b,
                W1x, W1y, W1a, W1u, b1, W2, b2, xn_out):
    Seac = Seap[0] + Seap[1] + Seaq[0] + Seaq[1]
    Sx = Sxp[0] + Sxp[1] + Sxq[0] + Sxq[1]
    Sx1 = Sx1p[0] + Sx1p[1] + Sx1q[0] + Sx1q[1]
    Sea = Seac[:, :EF]
    cnt = Seac[:, EF:EF + 1]
    cntc = jnp.maximum(cnt, 1.0)
    agg = (_dot(Sx, m1Wx[...]) + _dot(Sx1, m1Wy[...])
           + _dot(Sea, m1We[...]) + cnt * m1b[...]) / cntc
    oh = _onehot(b3[0, 0, :], B)
    ub = _dot(oh, uc[...])
    h = jnp.maximum(_dot(x[...], W1x[...]) + _dot(x1[...], W1y[...])
                    + _dot(agg, W1a[...]) + _dot(ub, W1u[...]) + b1[...], 0.0)
    xn_out[...] = _dot(h, W2[...]) + b2[...]


def _node2(x, x1, Sxp, Sxq, Sx1p, Sx1q, Seap, Seaq, b3, uc, m1Wx, m1Wy, m1We, m1b,
           W1x, W1y, W1a, W1u, b1, W2, b2):
    row = lambda i: (i, 0)
    row3 = lambda i: (0, i, 0)
    full = lambda i: (0, 0)
    return pl.pallas_call(
        _node2_body,
        grid=(NP // TN,),
        in_specs=[
            pl.BlockSpec((TN, NF), row), pl.BlockSpec((TN, NF), row),
            pl.BlockSpec((NC, TN, NF), row3), pl.BlockSpec((NC, TN, NF), row3),
            pl.BlockSpec((NC, TN, NF), row3), pl.BlockSpec((NC, TN, NF), row3),
            pl.BlockSpec((NC, TN, NF), row3), pl.BlockSpec((NC, TN, NF), row3),
            pl.BlockSpec((1, 1, TN), lambda i: (i, 0, 0)),
            pl.BlockSpec((B, 2 * GF), full),
            pl.BlockSpec((NF, H), full), pl.BlockSpec((NF, H), full),
            pl.BlockSpec((EF, H), full), pl.BlockSpec((1, H), full),
            pl.BlockSpec((NF, H), full), pl.BlockSpec((NF, H), full),
            pl.BlockSpec((H, H), full), pl.BlockSpec((2 * GF, H), full),
            pl.BlockSpec((1, H), full),
            pl.BlockSpec((H, NF), full), pl.BlockSpec((1, NF), full),
        ],
        out_specs=pl.BlockSpec((TN, NF), row),
        out_shape=jax.ShapeDtypeStruct((NP, NF), _f32),
    )(x, x1, Sxp, Sxq, Sx1p, Sx1q, Seap, Seaq, b3, uc, m1Wx, m1Wy, m1We, m1b,
      W1x, W1y, W1a, W1u, b1, W2, b2)


def _padn(a):
    return jnp.pad(a, ((0, NP - N),) + ((0, 0),) * (a.ndim - 1))


def kernel(x, edge_index, edge_attr, u, batch, e1_W1, e1_b1, e1_W2, e1_b2,
           n1_m1_W, n1_m1_b, n1_m2_W1, n1_m2_b1, n1_m2_W2, n1_m2_b2,
           g1_W1, g1_b1, g1_W2, g1_b2, e2_W1, e2_b1, e2_W2, e2_b2,
           n2_m1_W, n2_m1_b, n2_m2_W1, n2_m2_b1, n2_m2_W2, n2_m2_b2,
           g2_W1, g2_b1, g2_W2, g2_b2):
    row, col = edge_index[0], edge_index[1]
    rowp = jnp.pad(row, (0, EP - E))
    colp = jnp.pad(col, (0, EP - E), constant_values=NP - C)
    eattrp = jnp.pad(edge_attr, ((0, EP - E), (0, 0)))
    batchp = jnp.pad(batch, (0, NP - N), constant_values=B)
    zeros_nf = jnp.zeros((C, NF), _f32)
    xp = _padn(x)
    starts = jnp.searchsorted(batch, jnp.arange(B, dtype=_i32)).astype(_i32).reshape(1, B)
    ends = jnp.searchsorted(batch, jnp.arange(B, dtype=_i32), side="right").astype(_i32).reshape(1, B)
    r4 = rowp.reshape(NH, EP2 // TE, 1, TE)
    row4 = rowp.reshape(NH, NW, CPW, C)
    col4 = colp.reshape(NH, NW, CPW, C)
    ea4 = eattrp.reshape(NH, EP2, EF)

    ug1 = u @ e1_W1[2 * NF + EF:] + e1_b1
    e1w = (e1_W1[:NF], e1_W1[NF:2 * NF], e1_W1[2 * NF:2 * NF + EF], ug1,
           e1_W2, e1_b2.reshape(1, EF))
    xr, xc, Sxp, eaaug1, Seap1 = [], [], [], [], []
    for h in range(NH):
        a, b_, s = _sc_gather(xp, row4[h], col4[h], zeros_nf)
        xr.append(a); xc.append(b_); Sxp.append(s)
    for h in range(NH):
        eaaug1.append(_edge1(xr[h], xc[h], ea4[h], r4[h], starts, ends, *e1w))
        Seap1.append(_sc_scatter_ea(eaaug1[h], col4[h], zeros_nf))

    b3 = batchp.reshape(NP // TN, 1, TN)
    x1p, xsum = _node1(xp, Sxp[0], Sxp[1], Seap1[0], Seap1[1], b3, u,
                       n1_m1_W[:NF], n1_m1_W[NF:], n1_m1_b.reshape(1, H),
                       n1_m2_W1[:NF], n1_m2_W1[NF:NF + H], n1_m2_W1[NF + H:],
                       n1_m2_b1.reshape(1, H), n1_m2_W2, n1_m2_b2.reshape(1, NF))

    bcnt = jnp.maximum((ends - starts).astype(_f32).reshape(B), 1.0)
    xmean = xsum / bcnt[:, None]
    gh = jnp.maximum(jnp.concatenate([u, xmean], axis=1) @ g1_W1 + g1_b1, 0.0)
    u1 = gh @ g1_W2 + g1_b2
    uc = jnp.concatenate([u, u1], axis=1)

    ug2 = uc @ e2_W1[4 * NF + 2 * EF:] + e2_b1
    e2w = (e2_W1[:NF], e2_W1[NF:2 * NF], e2_W1[2 * NF:3 * NF], e2_W1[3 * NF:4 * NF],
           e2_W1[4 * NF:4 * NF + EF], e2_W1[4 * NF + EF:4 * NF + 2 * EF], ug2,
           e2_W2, e2_b2.reshape(1, EF))
    xr1, xc1, Sx1p, eaaug2, Seap2 = [], [], [], [], []
    for h in range(NH):
        a, b_, s = _sc_gather(x1p, row4[h], col4[h], zeros_nf)
        xr1.append(a); xc1.append(b_); Sx1p.append(s)
    for h in range(NH):
        eaaug2.append(_edge2(xr[h], xr1[h], xc[h], xc1[h], ea4[h], eaaug1[h],
                             r4[h], starts, ends, *e2w))
        Seap2.append(_sc_scatter_ea(eaaug2[h], col4[h], zeros_nf))

    x2p = _node2(xp, x1p, Sxp[0], Sxp[1], Sx1p[0], Sx1p[1], Seap2[0], Seap2[1],
                 b3, uc,
                 n2_m1_W[:NF], n2_m1_W[NF:2 * NF], n2_m1_W[2 * NF:],
                 n2_m1_b.reshape(1, H),
                 n2_m2_W1[:NF], n2_m2_W1[NF:2 * NF], n2_m2_W1[2 * NF:2 * NF + H],
                 n2_m2_W1[2 * NF + H:], n2_m2_b1.reshape(1, H),
                 n2_m2_W2, n2_m2_b2.reshape(1, NF))
    return x2p[:N]


# speedup vs baseline: 1.0708x; 1.0095x over previous
import functools

import jax
import jax.numpy as jnp
from jax import lax
from jax.experimental import pallas as pl
from jax.experimental.pallas import tpu as pltpu
from jax.experimental.pallas import tpu_sc as plsc

N = 10000
E = 160000
B = 16
NF = 128
EF = 16
GF = 16
H = 512

NC = 2
NS = 16
NW = NC * NS
C = 128
EP = 163840
NH = 4
EP2 = EP // NH
CPW = EP2 // NW // C
NP = 10240
STRIPE = NP // NS

NA = NF + B
TE = 1280
TN = 2048

_f32 = jnp.float32
_i32 = jnp.int32

_MESH = plsc.VectorSubcoreMesh(core_axis_name="c", subcore_axis_name="s")


def _sc_gather(xp, row3, col3, zeros_nf):
    @functools.partial(
        pl.kernel, mesh=_MESH,
        out_type=[
            jax.ShapeDtypeStruct((EP2, NF), _f32),
            jax.ShapeDtypeStruct((EP2, NF), _f32),
            jax.ShapeDtypeStruct((NC, NP, NF), _f32),
        ],
        scratch_types=[
            pltpu.VMEM((CPW, C), _i32), pltpu.VMEM((CPW, C), _i32),
            pltpu.VMEM((C, NF), _f32), pltpu.VMEM((C, NF), _f32),
            pltpu.VMEM_SHARED((NP, NF), _f32),
            pltpu.SemaphoreType.DMA, pltpu.SemaphoreType.DMA,
            pltpu.SemaphoreType.DMA, pltpu.SemaphoreType.DMA,
        ],
    )
    def k(x_h, row_h, col_h, z_h, xr_h, xc_h, sx_h,
          ri_v, ci_v, xr_v, xc_v, acc_s, sgr, sgc, swr, swc):
        cid = lax.axis_index("c")
        sid = lax.axis_index("s")
        wid = sid * NC + cid
        pltpu.sync_copy(z_h, xr_v)
        for z in range(STRIPE // C):
            pltpu.sync_copy(xr_v, acc_s.at[pl.ds(sid * STRIPE + z * C, C)])
        pltpu.sync_copy(row_h.at[wid], ri_v)
        pltpu.sync_copy(col_h.at[wid], ci_v)
        plsc.subcore_barrier()

        def fetch(off):
            pltpu.async_copy(x_h.at[ri_v.at[off]], xr_v, sgr)
            pltpu.async_copy(x_h.at[ci_v.at[off]], xc_v, sgc)

        fetch(0)

        def body(i, carry):
            base = wid * (EP2 // NW) + i * C
            pltpu.make_async_copy(x_h.at[ri_v.at[i]], xr_v, sgr).wait()
            wr = pltpu.async_copy(xr_v, xr_h.at[pl.ds(base, C)], swr)
            pltpu.sync_copy(xr_v, acc_s.at[ci_v.at[i]], add=True)
            pltpu.make_async_copy(x_h.at[ci_v.at[i]], xc_v, sgc).wait()
            wc = pltpu.async_copy(xc_v, xc_h.at[pl.ds(base, C)], swc)
            wr.wait()
            wc.wait()

            @pl.when(i + 1 < CPW)
            def _():
                fetch(i + 1)
            return carry

        lax.fori_loop(0, CPW, body, 0)
        plsc.subcore_barrier()
        pltpu.sync_copy(acc_s.at[pl.ds(sid * STRIPE, STRIPE)],
                        sx_h.at[cid, pl.ds(sid * STRIPE, STRIPE)])

    return k(xp, row3, col3, zeros_nf)


def _sc_scatter_ea(eaaug, col3, zeros_nf):
    @functools.partial(
        pl.kernel, mesh=_MESH,
        out_type=jax.ShapeDtypeStruct((NC, NP, NF), _f32),
        scratch_types=[
            pltpu.VMEM((CPW, C), _i32), pltpu.VMEM((2, C, NF), _f32),
            pltpu.VMEM_SHARED((NP, NF), _f32),
            pltpu.SemaphoreType.DMA, pltpu.SemaphoreType.DMA,
        ],
    )
    def k(ea_h, col_h, z_h, sea_h, ci_v, ea_v, acc_s, se0, se1):
        cid = lax.axis_index("c")
        sid = lax.axis_index("s")
        wid = sid * NC + cid
        ses = (se0, se1)
        pltpu.sync_copy(z_h, ea_v.at[0])
        for z in range(STRIPE // C):
            pltpu.sync_copy(ea_v.at[0], acc_s.at[pl.ds(sid * STRIPE + z * C, C)])
        pltpu.sync_copy(col_h.at[wid], ci_v)
        plsc.subcore_barrier()

        def fetch(off, b):
            base = wid * (EP2 // NW) + off * C
            pltpu.async_copy(ea_h.at[pl.ds(base, C)], ea_v.at[b], ses[b])

        def drain(off, b):
            base = wid * (EP2 // NW) + off * C
            pltpu.make_async_copy(ea_h.at[pl.ds(base, C)], ea_v.at[b], ses[b]).wait()

        fetch(0, 0)
        fetch(1, 1)

        def body(j, carry):
            for b in range(2):
                off = 2 * j + b
                drain(off, b)
                pltpu.sync_copy(ea_v.at[b], acc_s.at[ci_v.at[off]], add=True)

                @pl.when(off + 2 < CPW)
                def _():
                    fetch(off + 2, b)
            return carry

        lax.fori_loop(0, CPW // 2, body, 0)
        plsc.subcore_barrier()
        pltpu.sync_copy(acc_s.at[pl.ds(sid * STRIPE, STRIPE)],
                        sea_h.at[cid, pl.ds(sid * STRIPE, STRIPE)])

    return k(eaaug, col3, zeros_nf)


def _dot(a, b):
    return lax.dot_general(a, b, (((1,), (0,)), ((), ())), preferred_element_type=_f32)


def _dotb(a, b):
    return lax.dot_general(a.astype(jnp.bfloat16), b.astype(jnp.bfloat16),
                           (((1,), (0,)), ((), ())), preferred_element_type=_f32)


def _dott(a, b):
    return lax.dot_general(a, b, (((0,), (0,)), ((), ())), preferred_element_type=_f32)


def _onehot(idx, k):
    return (idx[:, None] == lax.broadcasted_iota(_i32, (idx.shape[0], k), 1)).astype(_f32)


def _bound_onehot(r3, starts, ends):
    rt = r3[0, 0, :][:, None]
    return ((rt >= starts[...]) & (rt < ends[...])).astype(_f32)


def _edge1_body(xr, xc, ea, r3, starts, ends, Ws, Wd, We, ug, W2, b2, out):
    oh = _bound_onehot(r3, starts, ends)
    h = (_dotb(xr[...], Ws[...]) + _dotb(xc[...], Wd[...])
         + _dot(ea[...], We[...]) + _dot(oh, ug[...]))
    eao = _dot(jnp.maximum(h, 0.0), W2[...]) + b2[...]
    out[...] = jnp.concatenate(
        [eao, jnp.ones((TE, 1), _f32), jnp.zeros((TE, NF - EF - 1), _f32)], axis=1)


def _edge1(xr, xc, eattr, r3, starts, ends, Ws, Wd, We, ug, W2, b2):
    row = lambda i: (i, 0)
    full = lambda i: (0, 0)
    return pl.pallas_call(
        _edge1_body,
        grid=(EP2 // TE,),
        in_specs=[
            pl.BlockSpec((TE, NF), row), pl.BlockSpec((TE, NF), row),
            pl.BlockSpec((TE, EF), row),
            pl.BlockSpec((1, 1, TE), lambda i: (i, 0, 0)),
            pl.BlockSpec((1, B), full), pl.BlockSpec((1, B), full),
            pl.BlockSpec((NF, H), full), pl.BlockSpec((NF, H), full),
            pl.BlockSpec((EF, H), full), pl.BlockSpec((B, H), full),
            pl.BlockSpec((H, EF), full), pl.BlockSpec((1, EF), full),
        ],
        out_specs=pl.BlockSpec((TE, NF), row),
        out_shape=jax.ShapeDtypeStruct((EP2, NF), _f32),
    )(xr, xc, eattr, r3, starts, ends, Ws, Wd, We, ug, W2, b2)


def _edge2_body(xr, xr1, xc, xc1, ea, eaaug1, r3, starts, ends,
                Wsx, Wsy, Wdx, Wdy, Wee, Wea, ug, W2, b2, out):
    oh = _bound_onehot(r3, starts, ends)
    h = (_dotb(xr[...], Wsx[...]) + _dotb(xr1[...], Wsy[...])
         + _dotb(xc[...], Wdx[...]) + _dotb(xc1[...], Wdy[...])
         + _dot(ea[...], Wee[...]) + _dot(eaaug1[:, :EF], Wea[...]) + _dot(oh, ug[...]))
    eao = _dot(jnp.maximum(h, 0.0), W2[...]) + b2[...]
    out[...] = jnp.concatenate(
        [eao, jnp.ones((TE, 1), _f32), jnp.zeros((TE, NF - EF - 1), _f32)], axis=1)


def _edge2(xr, xr1, xc, xc1, eattr, eaaug1, r3, starts, ends,
           Wsx, Wsy, Wdx, Wdy, Wee, Wea, ug, W2, b2):
    row = lambda i: (i, 0)
    full = lambda i: (0, 0)
    return pl.pallas_call(
        _edge2_body,
        grid=(EP2 // TE,),
        in_specs=[
            pl.BlockSpec((TE, NF), row), pl.BlockSpec((TE, NF), row),
            pl.BlockSpec((TE, NF), row), pl.BlockSpec((TE, NF), row),
            pl.BlockSpec((TE, EF), row), pl.BlockSpec((TE, NF), row),
            pl.BlockSpec((1, 1, TE), lambda i: (i, 0, 0)),
            pl.BlockSpec((1, B), full), pl.BlockSpec((1, B), full),
            pl.BlockSpec((NF, H), full), pl.BlockSpec((NF, H), full),
            pl.BlockSpec((NF, H), full), pl.BlockSpec((NF, H), full),
            pl.BlockSpec((EF, H), full), pl.BlockSpec((EF, H), full),
            pl.BlockSpec((B, H), full), pl.BlockSpec((H, EF), full),
            pl.BlockSpec((1, EF), full),
        ],
        out_specs=pl.BlockSpec((TE, NF), row),
        out_shape=jax.ShapeDtypeStruct((EP2, NF), _f32),
    )(xr, xr1, xc, xc1, eattr, eaaug1, r3, starts, ends,
      Wsx, Wsy, Wdx, Wdy, Wee, Wea, ug, W2, b2)


def _node1_body(x, Sxp, Sxq, Seap, Seaq, b3, u, m1Wx, m1We, m1b, W1x, W1a, W1u, b1, W2, b2,
                xn_out, xsum_out):
    i = pl.program_id(0)
    Seac = Seap[0] + Seap[1] + Seaq[0] + Seaq[1]
    Sx = Sxp[0] + Sxp[1] + Sxq[0] + Sxq[1]
    Sea = Seac[:, :EF]
    cnt = Seac[:, EF:EF + 1]
    cntc = jnp.maximum(cnt, 1.0)
    agg = (_dot(Sx, m1Wx[...]) + _dot(Sea, m1We[...]) + cnt * m1b[...]) / cntc
    oh = _onehot(b3[0, 0, :], B)
    ub = _dot(oh, u[...])
    h = jnp.maximum(_dot(x[...], W1x[...]) + _dot(agg, W1a[...]) + _dot(ub, W1u[...]) + b1[...], 0.0)
    xn = _dot(h, W2[...]) + b2[...]
    xn_out[...] = xn

    @pl.when(i == 0)
    def _():
        xsum_out[...] = jnp.zeros_like(xsum_out)

    xsum_out[...] += _dott(oh, xn)


def _node1(x, Sxp, Sxq, Seap, Seaq, b3, u, m1Wx, m1We, m1b, W1x, W1a, W1u, b1, W2, b2):
    row = lambda i: (i, 0)
    row3 = lambda i: (0, i, 0)
    full = lambda i: (0, 0)
    return pl.pallas_call(
        _node1_body,
        grid=(NP // TN,),
        in_specs=[
            pl.BlockSpec((TN, NF), row), pl.BlockSpec((NC, TN, NF), row3),
            pl.BlockSpec((NC, TN, NF), row3), pl.BlockSpec((NC, TN, NF), row3),
            pl.BlockSpec((NC, TN, NF), row3), pl.BlockSpec((1, 1, TN), lambda i: (i, 0, 0)),
            pl.BlockSpec((B, GF), full),
            pl.BlockSpec((NF, H), full), pl.BlockSpec((EF, H), full),
            pl.BlockSpec((1, H), full),
            pl.BlockSpec((NF, H), full), pl.BlockSpec((H, H), full),
            pl.BlockSpec((GF, H), full), pl.BlockSpec((1, H), full),
            pl.BlockSpec((H, NF), full), pl.BlockSpec((1, NF), full),
        ],
        out_specs=[pl.BlockSpec((TN, NF), row), pl.BlockSpec((B, NF), full)],
        out_shape=[jax.ShapeDtypeStruct((NP, NF), _f32),
                   jax.ShapeDtypeStruct((B, NF), _f32)],
    )(x, Sxp, Sxq, Seap, Seaq, b3, u, m1Wx, m1We, m1b, W1x, W1a, W1u, b1, W2, b2)


def _node2_body(x, x1, Sxp, Sxq, Sx1p, Sx1q, Seap, Seaq, b3, uc, m1Wx, m1Wy, m1We, m1b,
                W1x, W1y, W1a, W1u, b1, W2, b2, xn_out):
    Seac = Seap[0] + Seap[1] + Seaq[0] + Seaq[1]
    Sx = Sxp[0] + Sxp[1] + Sxq[0] + Sxq[1]
    Sx1 = Sx1p[0] + Sx1p[1] + Sx1q[0] + Sx1q[1]
    Sea = Seac[:, :EF]
    cnt = Seac[:, EF:EF + 1]
    cntc = jnp.maximum(cnt, 1.0)
    agg = (_dot(Sx, m1Wx[...]) + _dot(Sx1, m1Wy[...])
           + _dot(Sea, m1We[...]) + cnt * m1b[...]) / cntc
    oh = _onehot(b3[0, 0, :], B)
    ub = _dot(oh, uc[...])
    h = jnp.maximum(_dot(x[...], W1x[...]) + _dot(x1[...], W1y[...])
                    + _dot(agg, W1a[...]) + _dot(ub, W1u[...]) + b1[...], 0.0)
    xn_out[...] = _dot(h, W2[...]) + b2[...]


def _node2(x, x1, Sxp, Sxq, Sx1p, Sx1q, Seap, Seaq, b3, uc, m1Wx, m1Wy, m1We, m1b,
           W1x, W1y, W1a, W1u, b1, W2, b2):
    row = lambda i: (i, 0)
    row3 = lambda i: (0, i, 0)
    full = lambda i: (0, 0)
    return pl.pallas_call(
        _node2_body,
        grid=(NP // TN,),
        in_specs=[
            pl.BlockSpec((TN, NF), row), pl.BlockSpec((TN, NF), row),
            pl.BlockSpec((NC, TN, NF), row3), pl.BlockSpec((NC, TN, NF), row3),
            pl.BlockSpec((NC, TN, NF), row3), pl.BlockSpec((NC, TN, NF), row3),
            pl.BlockSpec((NC, TN, NF), row3), pl.BlockSpec((NC, TN, NF), row3),
            pl.BlockSpec((1, 1, TN), lambda i: (i, 0, 0)),
            pl.BlockSpec((B, 2 * GF), full),
            pl.BlockSpec((NF, H), full), pl.BlockSpec((NF, H), full),
            pl.BlockSpec((EF, H), full), pl.BlockSpec((1, H), full),
            pl.BlockSpec((NF, H), full), pl.BlockSpec((NF, H), full),
            pl.BlockSpec((H, H), full), pl.BlockSpec((2 * GF, H), full),
            pl.BlockSpec((1, H), full),
            pl.BlockSpec((H, NF), full), pl.BlockSpec((1, NF), full),
        ],
        out_specs=pl.BlockSpec((TN, NF), row),
        out_shape=jax.ShapeDtypeStruct((NP, NF), _f32),
    )(x, x1, Sxp, Sxq, Sx1p, Sx1q, Seap, Seaq, b3, uc, m1Wx, m1Wy, m1We, m1b,
      W1x, W1y, W1a, W1u, b1, W2, b2)


def _padn(a):
    return jnp.pad(a, ((0, NP - N),) + ((0, 0),) * (a.ndim - 1))


def kernel(x, edge_index, edge_attr, u, batch, e1_W1, e1_b1, e1_W2, e1_b2,
           n1_m1_W, n1_m1_b, n1_m2_W1, n1_m2_b1, n1_m2_W2, n1_m2_b2,
           g1_W1, g1_b1, g1_W2, g1_b2, e2_W1, e2_b1, e2_W2, e2_b2,
           n2_m1_W, n2_m1_b, n2_m2_W1, n2_m2_b1, n2_m2_W2, n2_m2_b2,
           g2_W1, g2_b1, g2_W2, g2_b2):
    row, col = edge_index[0], edge_index[1]
    rowp = jnp.pad(row, (0, EP - E))
    colp = jnp.pad(col, (0, EP - E), constant_values=NP - C)
    eattrp = jnp.pad(edge_attr, ((0, EP - E), (0, 0)))
    batchp = jnp.pad(batch, (0, NP - N), constant_values=B)
    zeros_nf = jnp.zeros((C, NF), _f32)
    xp = _padn(x)
    starts = jnp.searchsorted(batch, jnp.arange(B, dtype=_i32)).astype(_i32).reshape(1, B)
    ends = jnp.searchsorted(batch, jnp.arange(B, dtype=_i32), side="right").astype(_i32).reshape(1, B)
    r4 = rowp.reshape(NH, EP2 // TE, 1, TE)
    row4 = rowp.reshape(NH, NW, CPW, C)
    col4 = colp.reshape(NH, NW, CPW, C)
    ea4 = eattrp.reshape(NH, EP2, EF)

    ug1 = u @ e1_W1[2 * NF + EF:] + e1_b1
    e1w = (e1_W1[:NF], e1_W1[NF:2 * NF], e1_W1[2 * NF:2 * NF + EF], ug1,
           e1_W2, e1_b2.reshape(1, EF))
    xr, xc, Sxp, eaaug1, Seap1 = [], [], [], [], []
    for h in range(NH):
        a, b_, s = _sc_gather(xp, row4[h], col4[h], zeros_nf)
        xr.append(a); xc.append(b_); Sxp.append(s)
    for h in range(NH):
        eaaug1.append(_edge1(xr[h], xc[h], ea4[h], r4[h], starts, ends, *e1w))
        Seap1.append(_sc_scatter_ea(eaaug1[h], col4[h], zeros_nf))

    b3 = batchp.reshape(NP // TN, 1, TN)
    fold = lambda L: (sum(L[0::2][1:], L[0]), sum(L[1::2][1:], L[1]))
    SxA, SxB = fold(Sxp)
    Se1A, Se1B = fold(Seap1)
    x1p, xsum = _node1(xp, SxA, SxB, Se1A, Se1B, b3, u,
                       n1_m1_W[:NF], n1_m1_W[NF:], n1_m1_b.reshape(1, H),
                       n1_m2_W1[:NF], n1_m2_W1[NF:NF + H], n1_m2_W1[NF + H:],
                       n1_m2_b1.reshape(1, H), n1_m2_W2, n1_m2_b2.reshape(1, NF))

    bcnt = jnp.maximum((ends - starts).astype(_f32).reshape(B), 1.0)
    xmean = xsum / bcnt[:, None]
    gh = jnp.maximum(jnp.concatenate([u, xmean], axis=1) @ g1_W1 + g1_b1, 0.0)
    u1 = gh @ g1_W2 + g1_b2
    uc = jnp.concatenate([u, u1], axis=1)

    ug2 = uc @ e2_W1[4 * NF + 2 * EF:] + e2_b1
    e2w = (e2_W1[:NF], e2_W1[NF:2 * NF], e2_W1[2 * NF:3 * NF], e2_W1[3 * NF:4 * NF],
           e2_W1[4 * NF:4 * NF + EF], e2_W1[4 * NF + EF:4 * NF + 2 * EF], ug2,
           e2_W2, e2_b2.reshape(1, EF))
    xr1, xc1, Sx1p, eaaug2, Seap2 = [], [], [], [], []
    for h in range(NH):
        a, b_, s = _sc_gather(x1p, row4[h], col4[h], zeros_nf)
        xr1.append(a); xc1.append(b_); Sx1p.append(s)
    for h in range(NH):
        eaaug2.append(_edge2(xr[h], xr1[h], xc[h], xc1[h], ea4[h], eaaug1[h],
                             r4[h], starts, ends, *e2w))
        Seap2.append(_sc_scatter_ea(eaaug2[h], col4[h], zeros_nf))

    Sx1A, Sx1B = fold(Sx1p)
    Se2A, Se2B = fold(Seap2)
    x2p = _node2(xp, x1p, SxA, SxB, Sx1A, Sx1B, Se2A, Se2B,
                 b3, uc,
                 n2_m1_W[:NF], n2_m1_W[NF:2 * NF], n2_m1_W[2 * NF:],
                 n2_m1_b.reshape(1, H),
                 n2_m2_W1[:NF], n2_m2_W1[NF:2 * NF], n2_m2_W1[2 * NF:2 * NF + H],
                 n2_m2_W1[2 * NF + H:], n2_m2_b1.reshape(1, H),
                 n2_m2_W2, n2_m2_b2.reshape(1, NF))
    return x2p[:N]
